# baseline (speedup 1.0000x reference)
import functools

import jax
import jax.numpy as jnp
from jax import lax
from jax.experimental import pallas as pl
from jax.experimental.pallas import tpu as pltpu
from jax.experimental.pallas import tpu_sc as plsc

_L = 16
_UN = 4


@functools.cache
def _build(batch, feat_dim, num_classes):
    info = plsc.get_sparse_core_info()
    nc, ns = info.num_cores, info.num_subcores
    nw = nc * ns
    rows_per_w = feat_dim // nw
    scale = 0.5 / batch

    t_sz0 = (num_classes // 3) // 128 * 128
    t_lo = (0, t_sz0, 2 * t_sz0)
    t_sz = (t_sz0, t_sz0, num_classes - 2 * t_sz0)
    t_max = max(t_sz)
    n_seg = 3
    n_asn = rows_per_w * n_seg
    pk_pad = batch + 4 * _L

    mesh = plsc.VectorSubcoreMesh(core_axis_name="c", subcore_axis_name="s")

    @functools.partial(
        pl.kernel,
        out_type=jax.ShapeDtypeStruct((nw, _L), jnp.float32),
        mesh=mesh,
        compiler_params=pltpu.CompilerParams(
            use_tc_tiling_on_sc=True, needs_layout_passes=False),
        scratch_types=[
            pltpu.VMEM((t_max,), jnp.float32),
            pltpu.VMEM((t_max,), jnp.float32),
            pltpu.VMEM((batch,), jnp.int32),
            pltpu.VMEM((pk_pad,), jnp.int32),
            pltpu.VMEM((batch,), jnp.float32),
            pltpu.VMEM((_L,), jnp.float32),
            pltpu.SemaphoreType.DMA,
            pltpu.SemaphoreType.DMA,
            pltpu.SemaphoreType.DMA,
        ],
    )
    def k(featT_hbm, lab_hbm, centT_hbm, out_hbm, rowA, rowB, lab_v, pk_v,
          feat_v, acc_v, semA, semB, fsem):
        wid = lax.axis_index("s") * nc + lax.axis_index("c")
        f0 = wid * rows_per_w
        bufs = (rowA, rowB)
        sems = (semA, semB)

        def row_copy(j, buf, sem):
            r, t = j // n_seg, j % n_seg
            return pltpu.async_copy(
                centT_hbm.at[f0 + r].at[pl.ds(t_lo[t], t_sz[t])],
                buf.at[pl.ds(0, t_sz[t])], sem)

        copies = [None] * n_asn
        copies[0] = row_copy(0, bufs[0], sems[0])
        copies[1] = row_copy(1, bufs[1], sems[1])
        fcopy = pltpu.async_copy(featT_hbm.at[f0], feat_v, fsem)
        pltpu.sync_copy(lab_hbm, lab_v)

        zeros16 = jnp.zeros((_L,), jnp.int32)
        for p in range(batch, pk_pad, _L):
            pk_v[pl.ds(p, _L)] = zeros16
        iota16 = lax.iota(jnp.int32, _L)

        def cbody(g, carry):
            v1, v2 = carry
            l = lab_v[pl.ds(g * _L, _L)]
            v1 = v1 + (l < t_lo[1]).astype(jnp.int32)
            v2 = v2 + (l < t_lo[2]).astype(jnp.int32)
            return v1, v2

        v1, v2 = lax.fori_loop(0, batch // _L, cbody, (zeros16, zeros16))
        n0 = jnp.sum(v1)
        n12 = jnp.sum(v2)

        def fbody(g, carry):
            o0, o1, o2 = carry
            l = lab_v[pl.ds(g * _L, _L)]
            org = iota16 + g * _L
            pk = lax.bitwise_or(lax.shift_left(org, 17), l)
            m0 = l < t_lo[1]
            m2 = l >= t_lo[2]
            m1 = jnp.logical_not(jnp.logical_or(m0, m2))
            plsc.store_compressed(pk_v.at[pl.ds(o0, _L)], pk, mask=m0)
            plsc.store_compressed(pk_v.at[pl.ds(o1, _L)], pk, mask=m1)
            plsc.store_compressed(pk_v.at[pl.ds(o2, _L)], pk, mask=m2)
            o0 = o0 + jnp.sum(m0.astype(jnp.int32))
            o1 = o1 + jnp.sum(m1.astype(jnp.int32))
            o2 = o2 + jnp.sum(m2.astype(jnp.int32))
            return o0, o1, o2

        lax.fori_loop(0, batch // _L, fbody, (0, n0, n12))

        seg_base = (0, n0, n12)
        seg_n = (n0, n12 - n0, batch - n12)
        fcopy.wait()

        acc = jnp.zeros((_L,), jnp.float32)
        for j in range(n_asn):
            r, t = j // n_seg, j % n_seg
            copies[j].wait()
            if t == 0 and r > 0:
                pltpu.sync_copy(featT_hbm.at[f0 + r], feat_v)
            row = bufs[j % 2]
            base, n = seg_base[t], seg_n[t]
            lim = t_sz[t] - 1

            def sbody(i, acc):
                for u in range(_UN):
                    k0 = base + i * (_UN * _L) + u * _L
                    pk = pk_v[pl.ds(k0, _L)]
                    org = lax.shift_right_logical(pk, 17)
                    l = lax.bitwise_and(pk, 0x1FFFF)
                    ci = jnp.clip(l - t_lo[t], 0, lim)
                    c = plsc.load_gather(row, [ci])
                    fv = plsc.load_gather(feat_v, [org])
                    d = fv - c
                    valid = (iota16 + (i * (_UN * _L) + u * _L)) < n
                    acc = acc + jnp.where(valid, d * d, 0.0)
                return acc

            ng = (n + _UN * _L - 1) // (_UN * _L)
            acc = lax.fori_loop(0, ng, sbody, acc)
            if j + 2 < n_asn:
                copies[j + 2] = row_copy(j + 2, bufs[j % 2], sems[j % 2])

        acc_v[...] = acc * scale
        pltpu.sync_copy(acc_v, out_hbm.at[wid])

    return k


def kernel(features, labels, centers):
    batch, feat_dim = features.shape
    num_classes = centers.shape[0]
    k = _build(batch, feat_dim, num_classes)
    partials = k(features.T, labels.astype(jnp.int32), centers.T)
    return jnp.sum(partials)


# speedup vs baseline: 1.2864x; 1.2864x over previous
import functools

import jax
import jax.numpy as jnp
from jax import lax
from jax.experimental import pallas as pl
from jax.experimental.pallas import tpu as pltpu
from jax.experimental.pallas import tpu_sc as plsc

_L = 16
_UNROLL = 16


@functools.cache
def _build(batch, feat_dim, num_classes):
    info = plsc.get_sparse_core_info()
    nc, ns = info.num_cores, info.num_subcores
    nw = nc * ns
    rows_per_w = feat_dim // nw
    f_chunk = 4096
    n_fchunk = batch // f_chunk
    scale = 0.5 / batch

    mesh = plsc.VectorSubcoreMesh(core_axis_name="c", subcore_axis_name="s")

    @functools.partial(
        pl.kernel,
        out_type=jax.ShapeDtypeStruct((nw, _L), jnp.float32),
        mesh=mesh,
        compiler_params=pltpu.CompilerParams(
            use_tc_tiling_on_sc=True, needs_layout_passes=False),
        scratch_types=[
            pltpu.VMEM((num_classes,), jnp.float32),
            pltpu.VMEM((batch,), jnp.int32),
            pltpu.VMEM((f_chunk,), jnp.float32),
            pltpu.VMEM((f_chunk,), jnp.float32),
            pltpu.VMEM((_L,), jnp.float32),
            pltpu.SemaphoreType.DMA,
            pltpu.SemaphoreType.DMA,
            pltpu.SemaphoreType.DMA,
        ],
    )
    def k(featT_hbm, lab_hbm, centT_hbm, out_hbm, row_v, lab_v, feat_v0,
          feat_v1, acc_v, rsem, fsem0, fsem1):
        wid = lax.axis_index("s") * nc + lax.axis_index("c")

        rcopy = pltpu.async_copy(centT_hbm.at[wid * rows_per_w], row_v, rsem)
        pltpu.sync_copy(lab_hbm, lab_v)

        fbufs = (feat_v0, feat_v1)
        fsems = (fsem0, fsem1)

        acc = jnp.zeros((_L,), jnp.float32)
        for r in range(rows_per_w):
            f = wid * rows_per_w + r
            fcopies = [None] * n_fchunk
            fcopies[0] = pltpu.async_copy(
                featT_hbm.at[f, pl.ds(0, f_chunk)], fbufs[0], fsems[0])
            rcopy.wait()
            for h in range(n_fchunk):
                if h + 1 < n_fchunk:
                    b = (h + 1) % 2
                    fcopies[h + 1] = pltpu.async_copy(
                        featT_hbm.at[f, pl.ds((h + 1) * f_chunk, f_chunk)],
                        fbufs[b], fsems[b])
                fcopies[h].wait()
                feat_v = fbufs[h % 2]

                def body(it, acc):
                    base = it * (_L * _UNROLL)
                    for u in range(_UNROLL):
                        o = base + u * _L
                        idx = lab_v[pl.ds(h * f_chunk + o, _L)]
                        c = plsc.load_gather(row_v, [idx])
                        fv = feat_v[pl.ds(o, _L)]
                        d = fv - c
                        acc = acc + d * d
                    return acc

                acc = lax.fori_loop(0, f_chunk // (_L * _UNROLL), body, acc)
            if r < rows_per_w - 1:
                rcopy = pltpu.async_copy(
                    centT_hbm.at[wid * rows_per_w + r + 1], row_v, rsem)

        acc_v[...] = acc * scale
        pltpu.sync_copy(acc_v, out_hbm.at[wid])

    return k


def kernel(features, labels, centers):
    batch, feat_dim = features.shape
    num_classes = centers.shape[0]
    k = _build(batch, feat_dim, num_classes)
    partials = k(features.T, labels.astype(jnp.int32), centers.T)
    return jnp.sum(partials)


# speedup vs baseline: 1.3286x; 1.0328x over previous
import functools

import jax
import jax.numpy as jnp
from jax import lax
from jax.experimental import pallas as pl
from jax.experimental.pallas import tpu as pltpu
from jax.experimental.pallas import tpu_sc as plsc

_L = 16
_UNROLL = 8


@functools.cache
def _build(batch, feat_dim, num_classes):
    info = plsc.get_sparse_core_info()
    nc, ns = info.num_cores, info.num_subcores
    nw = nc * ns
    rows_per_w = feat_dim // nw
    f_chunk = 4096
    n_fchunk = batch // f_chunk
    scale = 0.5 / batch

    mesh = plsc.VectorSubcoreMesh(core_axis_name="c", subcore_axis_name="s")

    @functools.partial(
        pl.kernel,
        out_type=jax.ShapeDtypeStruct((nw, _L), jnp.float32),
        mesh=mesh,
        compiler_params=pltpu.CompilerParams(
            use_tc_tiling_on_sc=True, needs_layout_passes=False),
        scratch_types=[
            pltpu.VMEM((num_classes,), jnp.float32),
            pltpu.VMEM((batch,), jnp.int32),
            pltpu.VMEM((f_chunk,), jnp.float32),
            pltpu.VMEM((f_chunk,), jnp.float32),
            pltpu.VMEM((_L,), jnp.float32),
            pltpu.SemaphoreType.DMA,
            pltpu.SemaphoreType.DMA,
            pltpu.SemaphoreType.DMA,
        ],
    )
    def k(featT_hbm, lab_hbm, centT_hbm, out_hbm, row_v, lab_v, feat_v0,
          feat_v1, acc_v, rsem, fsem0, fsem1):
        wid = lax.axis_index("s") * nc + lax.axis_index("c")

        rcopy = pltpu.async_copy(centT_hbm.at[wid * rows_per_w], row_v, rsem)
        pltpu.sync_copy(lab_hbm, lab_v)

        fbufs = (feat_v0, feat_v1)
        fsems = (fsem0, fsem1)

        acc = jnp.zeros((_L,), jnp.float32)
        for r in range(rows_per_w):
            f = wid * rows_per_w + r
            fcopies = [None] * n_fchunk
            fcopies[0] = pltpu.async_copy(
                featT_hbm.at[f, pl.ds(0, f_chunk)], fbufs[0], fsems[0])
            rcopy.wait()
            for h in range(n_fchunk):
                if h + 1 < n_fchunk:
                    b = (h + 1) % 2
                    fcopies[h + 1] = pltpu.async_copy(
                        featT_hbm.at[f, pl.ds((h + 1) * f_chunk, f_chunk)],
                        fbufs[b], fsems[b])
                fcopies[h].wait()
                feat_v = fbufs[h % 2]

                def body(it, acc):
                    base = it * (_L * _UNROLL)
                    for u in range(_UNROLL):
                        o = base + u * _L
                        idx = lab_v[pl.ds(h * f_chunk + o, _L)]
                        c = plsc.load_gather(row_v, [idx])
                        fv = feat_v[pl.ds(o, _L)]
                        d = fv - c
                        acc = acc + d * d
                    return acc

                acc = lax.fori_loop(0, f_chunk // (_L * _UNROLL), body, acc)
            if r < rows_per_w - 1:
                rcopy = pltpu.async_copy(
                    centT_hbm.at[wid * rows_per_w + r + 1], row_v, rsem)

        acc_v[...] = acc * scale
        pltpu.sync_copy(acc_v, out_hbm.at[wid])

    return k


def kernel(features, labels, centers):
    batch, feat_dim = features.shape
    num_classes = centers.shape[0]
    k = _build(batch, feat_dim, num_classes)
    partials = k(features.T, labels.astype(jnp.int32), centers.T)
    return jnp.sum(partials)


# speedup vs baseline: 1.3438x; 1.0114x over previous
import functools

import jax
import jax.numpy as jnp
from jax import lax
from jax.experimental import pallas as pl
from jax.experimental.pallas import tpu as pltpu
from jax.experimental.pallas import tpu_sc as plsc

_L = 16
_UNROLL = 8


@functools.cache
def _build(batch, feat_dim, num_classes):
    info = plsc.get_sparse_core_info()
    nc, ns = info.num_cores, info.num_subcores
    nw = nc * ns
    rows_per_w = feat_dim // nw
    f_chunk = 4096
    n_fchunk = batch // f_chunk
    scale = 0.5 / batch

    mesh = plsc.VectorSubcoreMesh(core_axis_name="c", subcore_axis_name="s")

    @functools.partial(
        pl.kernel,
        out_type=jax.ShapeDtypeStruct((nw, _L), jnp.float32),
        mesh=mesh,
        compiler_params=pltpu.CompilerParams(
            use_tc_tiling_on_sc=True, needs_layout_passes=False),
        scratch_types=[
            pltpu.VMEM((num_classes,), jnp.float32),
            pltpu.VMEM((batch,), jnp.int32),
            pltpu.VMEM((f_chunk,), jnp.float32),
            pltpu.VMEM((f_chunk,), jnp.float32),
            pltpu.VMEM((_L,), jnp.float32),
            pltpu.SemaphoreType.DMA,
            pltpu.SemaphoreType.DMA,
            pltpu.SemaphoreType.DMA,
        ],
    )
    def k(featT_hbm, lab_hbm, centT_hbm, out_hbm, row_v, lab_v, feat_v0,
          feat_v1, acc_v, rsem, fsem0, fsem1):
        wid = lax.axis_index("s") * nc + lax.axis_index("c")

        rcopy = pltpu.async_copy(centT_hbm.at[wid * rows_per_w], row_v, rsem)
        pltpu.sync_copy(lab_hbm, lab_v)

        fbufs = (feat_v0, feat_v1)
        fsems = (fsem0, fsem1)

        acc = (jnp.zeros((_L,), jnp.float32),) * 4
        for r in range(rows_per_w):
            f = wid * rows_per_w + r
            fcopies = [None] * n_fchunk
            fcopies[0] = pltpu.async_copy(
                featT_hbm.at[f, pl.ds(0, f_chunk)], fbufs[0], fsems[0])
            rcopy.wait()
            for h in range(n_fchunk):
                if h + 1 < n_fchunk:
                    b = (h + 1) % 2
                    fcopies[h + 1] = pltpu.async_copy(
                        featT_hbm.at[f, pl.ds((h + 1) * f_chunk, f_chunk)],
                        fbufs[b], fsems[b])
                fcopies[h].wait()
                feat_v = fbufs[h % 2]

                def body(it, acc):
                    base = it * (_L * _UNROLL)
                    acc = list(acc)
                    for u in range(_UNROLL):
                        o = base + u * _L
                        idx = lab_v[pl.ds(h * f_chunk + o, _L)]
                        c = plsc.load_gather(row_v, [idx])
                        fv = feat_v[pl.ds(o, _L)]
                        d = fv - c
                        acc[u % 4] = acc[u % 4] + d * d
                    return tuple(acc)

                acc = lax.fori_loop(0, f_chunk // (_L * _UNROLL), body, acc)
            if r < rows_per_w - 1:
                rcopy = pltpu.async_copy(
                    centT_hbm.at[wid * rows_per_w + r + 1], row_v, rsem)

        acc_v[...] = (acc[0] + acc[1] + (acc[2] + acc[3])) * scale
        pltpu.sync_copy(acc_v, out_hbm.at[wid])

    return k


def kernel(features, labels, centers):
    batch, feat_dim = features.shape
    num_classes = centers.shape[0]
    k = _build(batch, feat_dim, num_classes)
    partials = k(features.T, labels.astype(jnp.int32), centers.T)
    return jnp.sum(partials)
